# baseline (speedup 1.0000x reference)
import functools

import jax
import jax.numpy as jnp
from jax import lax
from jax.experimental import pallas as pl
from jax.experimental.pallas import tpu as pltpu
from jax.experimental.pallas import tpu_sc as plsc

_NC = 2
_NS = 16
_NW = _NC * _NS
_C = 64
_NBUF = 12


@functools.lru_cache(maxsize=None)
def _make_gather(b: int, l: int, dim: int):
    mesh = plsc.VectorSubcoreMesh(
        core_axis_name="c", subcore_axis_name="s", num_cores=_NC
    )
    kb = b // _C
    n_chunks = l * kb // _NW

    @functools.partial(
        pl.kernel,
        mesh=mesh,
        out_type=jax.ShapeDtypeStruct((l, b, dim), jnp.float32),
        scratch_types=[
            pltpu.VMEM((n_chunks, _C), jnp.int32),
            pltpu.VMEM((_NBUF, _C, dim), jnp.float32),
            pltpu.SemaphoreType.DMA((_NBUF,)),
            pltpu.SemaphoreType.DMA((_NBUF,)),
        ],
    )
    def k(table_hbm, idx_hbm, out_hbm, idx_v, rows_v, gsem, wsem):
        wid = lax.axis_index("s") * _NC + lax.axis_index("c")
        base = wid * n_chunks
        pltpu.sync_copy(idx_hbm.at[wid], idx_v)

        def chunk_out(j):
            c = base + j
            return out_hbm.at[lax.div(c, kb), pl.ds(lax.rem(c, kb) * _C, _C)]

        for u in range(_NBUF - 1):
            pltpu.async_copy(table_hbm.at[idx_v.at[u]], rows_v.at[u], gsem.at[u])

        def body(j, carry):
            bb = lax.rem(j, _NBUF)
            pltpu.make_async_copy(
                table_hbm.at[idx_v.at[j]], rows_v.at[bb], gsem.at[bb]
            ).wait()
            pltpu.async_copy(rows_v.at[bb], chunk_out(j), wsem.at[bb])

            bn = lax.rem(j + _NBUF - 1, _NBUF)

            @pl.when(j + _NBUF - 1 < n_chunks)
            def _():
                @pl.when(j >= 1)
                def _():
                    pltpu.make_async_copy(
                        rows_v.at[bn], chunk_out(j - 1), wsem.at[bn]
                    ).wait()

                pltpu.async_copy(
                    table_hbm.at[idx_v.at[j + _NBUF - 1]],
                    rows_v.at[bn],
                    gsem.at[bn],
                )

            return carry

        lax.fori_loop(0, n_chunks, body, 0)

        for t in range(_NBUF):
            j = n_chunks - _NBUF + t
            pltpu.make_async_copy(
                rows_v.at[j % _NBUF], chunk_out(j), wsem.at[j % _NBUF]
            ).wait()

    return k


def kernel(index, table):
    b, l = index.shape
    dim = table.shape[1]
    assert b % _C == 0 and (l * b // _C) % _NW == 0
    idx_c = jnp.transpose(index).reshape(_NW, l * b // _C // _NW, _C)
    out_p = _make_gather(b, l, dim)(table, idx_c)
    return jnp.transpose(out_p, (1, 0, 2))
